# baseline (speedup 1.0000x reference)
import functools

import jax
import jax.numpy as jnp
from jax import lax
from jax.experimental import pallas as pl
from jax.experimental.pallas import tpu as pltpu
from jax.experimental.pallas import tpu_sc as plsc

_N_E = 8192
_E_DIM = 32
_BETA = 0.25
_ZT = 512
_CB_CHUNK = 2048


def _argmin_body(z_ref, cb_ref, idx_ref, loss_ref):
    z = z_ref[...]
    znorm = jnp.sum(z * z, axis=1, keepdims=True)
    zm2 = z * (-2.0)

    mvl = jnp.full((_ZT, 128), jnp.inf, dtype=jnp.float32)
    mbl = jnp.zeros((_ZT, 128), dtype=jnp.int32)
    for c in range(_N_E // _CB_CHUNK):
        cb = cb_ref[pl.ds(c * _CB_CHUNK, _CB_CHUNK), :]
        cn = jnp.sum(cb * cb, axis=1)[None, :]
        dot = lax.dot_general(zm2, cb, (((1,), (1,)), ((), ())),
                              preferred_element_type=jnp.float32)
        s = (znorm + cn) + dot
        for b in range(_CB_CHUNK // 128):
            sb = s[:, b * 128:(b + 1) * 128]
            blk = c * (_CB_CHUNK // 128) + b
            upd = sb < mvl
            mvl = jnp.minimum(sb, mvl)
            mbl = jnp.where(upd, blk, mbl)

    jfull = mbl * 128 + lax.broadcasted_iota(jnp.int32, (_ZT, 128), 1)
    m = jnp.min(mvl, axis=1, keepdims=True)
    mi = jnp.min(jnp.where(mvl == m, jfull, _N_E), axis=1, keepdims=True)
    idx_ref[...] = mi
    loss_ref[...] = jnp.broadcast_to(jnp.sum(m, axis=0, keepdims=True),
                                     (1, 128))[None]


def _tc_argmin(z_flat, codebook, interpret=False):
    n = z_flat.shape[0]
    grid = n // _ZT
    return pl.pallas_call(
        _argmin_body,
        grid=(grid,),
        in_specs=[
            pl.BlockSpec((_ZT, _E_DIM), lambda i: (i, 0)),
            pl.BlockSpec((_N_E, _E_DIM), lambda i: (0, 0)),
        ],
        out_specs=[
            pl.BlockSpec((_ZT, 1), lambda i: (i, 0)),
            pl.BlockSpec((1, 1, 128), lambda i: (i, 0, 0)),
        ],
        out_shape=[
            jax.ShapeDtypeStruct((n, 1), jnp.int32),
            jax.ShapeDtypeStruct((grid, 1, 128), jnp.float32),
        ],
        interpret=interpret,
    )(z_flat, codebook)


def _sc_gather(codebook, idx):
    b = idx.shape[0]
    info = plsc.get_sparse_core_info()
    nw = info.num_cores * info.num_subcores
    bpw = b // nw
    mesh = plsc.VectorSubcoreMesh(core_axis_name="c", subcore_axis_name="s")

    @functools.partial(
        pl.kernel,
        out_type=jax.ShapeDtypeStruct((b, _E_DIM), jnp.float32),
        mesh=mesh,
        scratch_types=[
            pltpu.VMEM((bpw,), jnp.int32),
            pltpu.VMEM((bpw, _E_DIM), jnp.float32),
            pltpu.SemaphoreType.DMA,
        ],
        compiler_params=pltpu.CompilerParams(use_tc_tiling_on_sc=False),
    )
    def gather_k(table_hbm, idx_hbm, out_hbm, idx_v, rows_v, sem):
        wid = lax.axis_index("s") * info.num_cores + lax.axis_index("c")
        base = wid * bpw
        pltpu.sync_copy(idx_hbm.at[pl.ds(base, bpw)], idx_v)
        pltpu.async_copy(table_hbm.at[idx_v], rows_v, sem).wait()
        pltpu.sync_copy(rows_v, out_hbm.at[pl.ds(base, bpw)])

    return gather_k(codebook, idx)


def kernel(z, codebook):
    b, c, h, w = z.shape
    z_flat = jnp.transpose(z, (0, 2, 3, 1)).reshape(-1, _E_DIM)
    n = z_flat.shape[0]

    idx2d, loss_parts = _tc_argmin(z_flat, codebook)
    idx = idx2d.reshape(-1)

    z_q = _sc_gather(codebook, idx)

    loss = jnp.sum(loss_parts[:, 0, 0]) * ((1.0 + _BETA) / (n * _E_DIM))
    z_q_out = z_q.reshape(b, h, w, c)
    indices_out = idx.reshape(b, 1, h, w)
    return z_q_out, loss, indices_out


# speedup vs baseline: 1.0829x; 1.0829x over previous
import functools

import jax
import jax.numpy as jnp
from jax import lax
from jax.experimental import pallas as pl
from jax.experimental.pallas import tpu as pltpu
from jax.experimental.pallas import tpu_sc as plsc

_N_E = 8192
_E_DIM = 32
_BETA = 0.25
_HW = 1024
_CB_CHUNK = 2048
_SLICE = 128


def _argmin_body(z_ref, cb_ref, idx_ref, loss_ref):
    zb = z_ref[0]
    znorm = jnp.sum(zb * zb, axis=0, keepdims=True)
    zm2 = zb * (-2.0)

    mv = jnp.full((_SLICE, _HW), jnp.inf, dtype=jnp.float32)
    mbl = jnp.zeros((_SLICE, _HW), dtype=jnp.int32)
    for c in range(_N_E // _CB_CHUNK):
        cb = cb_ref[pl.ds(c * _CB_CHUNK, _CB_CHUNK), :]
        cn = jnp.sum(cb * cb, axis=1, keepdims=True)
        dot = lax.dot_general(cb, zm2, (((1,), (0,)), ((), ())),
                              preferred_element_type=jnp.float32)
        s = (znorm + cn) + dot
        for b in range(_CB_CHUNK // _SLICE):
            sb = s[b * _SLICE:(b + 1) * _SLICE, :]
            blk = c * (_CB_CHUNK // _SLICE) + b
            upd = sb < mv
            mv = jnp.minimum(sb, mv)
            mbl = jnp.where(upd, blk, mbl)

    jfull = mbl * _SLICE + lax.broadcasted_iota(jnp.int32, (_SLICE, _HW), 0)
    m = jnp.min(mv, axis=0, keepdims=True)
    mi = jnp.min(jnp.where(mv == m, jfull, _N_E), axis=0, keepdims=True)
    idx_ref[...] = mi[None]

    part = jnp.sum(m)
    i = pl.program_id(0)

    @pl.when(i == 0)
    def _():
        loss_ref[0, 0] = part

    @pl.when(i > 0)
    def _():
        loss_ref[0, 0] += part

    @pl.when(i == pl.num_programs(0) - 1)
    def _():
        loss_ref[0, 0] *= (1.0 + _BETA) / (4 * _HW * _E_DIM)


def _tc_argmin(z3, codebook):
    nb = z3.shape[0]
    return pl.pallas_call(
        _argmin_body,
        grid=(nb,),
        in_specs=[
            pl.BlockSpec((1, _E_DIM, _HW), lambda i: (i, 0, 0)),
            pl.BlockSpec((_N_E, _E_DIM), lambda i: (0, 0)),
        ],
        out_specs=[
            pl.BlockSpec((1, 1, _HW), lambda i: (i, 0, 0)),
            pl.BlockSpec(memory_space=pltpu.SMEM),
        ],
        out_shape=[
            jax.ShapeDtypeStruct((nb, 1, _HW), jnp.int32),
            jax.ShapeDtypeStruct((1, 1), jnp.float32),
        ],
    )(z3, codebook)


def _sc_gather(codebook, idx):
    b = idx.shape[0]
    info = plsc.get_sparse_core_info()
    nw = info.num_cores * info.num_subcores
    bpw = b // nw
    mesh = plsc.VectorSubcoreMesh(core_axis_name="c", subcore_axis_name="s")

    @functools.partial(
        pl.kernel,
        out_type=jax.ShapeDtypeStruct((b, _E_DIM), jnp.float32),
        mesh=mesh,
        scratch_types=[
            pltpu.VMEM((bpw,), jnp.int32),
            pltpu.VMEM((bpw, _E_DIM), jnp.float32),
            pltpu.SemaphoreType.DMA,
        ],
        compiler_params=pltpu.CompilerParams(use_tc_tiling_on_sc=False),
    )
    def gather_k(table_hbm, idx_hbm, out_hbm, idx_v, rows_v, sem):
        wid = lax.axis_index("s") * info.num_cores + lax.axis_index("c")
        base = wid * bpw
        pltpu.sync_copy(idx_hbm.at[pl.ds(base, bpw)], idx_v)
        pltpu.async_copy(table_hbm.at[idx_v], rows_v, sem).wait()
        pltpu.sync_copy(rows_v, out_hbm.at[pl.ds(base, bpw)])

    return gather_k(codebook, idx)


def kernel(z, codebook):
    b, c, h, w = z.shape
    z3 = z.reshape(b, c, h * w)

    idx3d, loss2d = _tc_argmin(z3, codebook)
    idx = idx3d.reshape(-1)

    z_q = _sc_gather(codebook, idx)

    loss = loss2d.reshape(())
    z_q_out = jnp.transpose(z_q.reshape(b, h, w, c), (0, 3, 1, 2))
    indices_out = idx3d.reshape(b, 1, h, w)
    return z_q_out, loss, indices_out
